# baseline (speedup 1.0000x reference)
import jax
import jax.numpy as jnp
from jax.experimental import pallas as pl
from jax.experimental.pallas import tpu as pltpu
from jax.experimental.pallas import tpu_sc as plsc

_H, _D, _KV = 4, 256, 4096
_ROWS = _H * _D
_TILES = 32
_RPT = _ROWS // _TILES
_CROWS = 16


def _sc_v3_body(pos_hbm, vs1_hbm, vs2_hbm, vs3_hbm, vc1, vc2, vc3,
                o1, o2, o3, posm, sl1, sl2, sl3, buf):
    cid = jax.lax.axis_index("c")
    sid = jax.lax.axis_index("s")
    wid = sid * 2 + cid
    pltpu.sync_copy(pos_hbm, posm)
    pltpu.sync_copy(vs1_hbm, sl1)
    pltpu.sync_copy(vs2_hbm, sl2)
    pltpu.sync_copy(vs3_hbm, sl3)
    p = posm[...][0]
    a16 = pl.multiple_of((p // 16) * 16, 8)
    m16 = p - a16
    ji = jax.lax.iota(jnp.int32, 16)
    for vc, sl, o in ((vc1, sl1, o1), (vc2, sl2, o2), (vc3, sl3, o3)):
        for c in range(_RPT // _CROWS):
            r = wid * _RPT + c * _CROWS
            h = r // _D
            rr = r - h * _D
            pltpu.sync_copy(vc.at[0, h, pl.ds(rr, _CROWS), :], buf)
            vals = sl[pl.ds(r, _CROWS)]
            for j in range(_CROWS):
                w = buf[j, pl.ds(a16, 16)]
                buf[j, pl.ds(a16, 16)] = jnp.where(ji == m16, vals[j], w)
            pltpu.sync_copy(buf, o.at[0, h, pl.ds(rr, _CROWS), :])


def _sc_update_v_caches(input_pos, vc1, vs1, vc2, vs2, vc3, vs3):
    run = pl.kernel(
        _sc_v3_body,
        out_type=tuple(jax.ShapeDtypeStruct(c.shape, c.dtype)
                       for c in (vc1, vc2, vc3)),
        mesh=plsc.VectorSubcoreMesh(core_axis_name="c", subcore_axis_name="s"),
        scratch_types=[
            pltpu.VMEM((16,), jnp.int32),
            pltpu.VMEM((_ROWS,), jnp.float32),
            pltpu.VMEM((_ROWS,), jnp.float32),
            pltpu.VMEM((_ROWS,), jnp.float32),
            pltpu.VMEM((_CROWS, _KV), jnp.float32),
        ],
    )
    pos16 = jnp.broadcast_to(input_pos.astype(jnp.int32), (16,))
    return run(pos16, vs1.reshape(-1), vs2.reshape(-1),
               vs3.reshape(-1), vc1, vc2, vc3)


def _scatter_body(pos_ref,
                  c0, c1, c2, c3, c4,
                  ks0, ks1, ks2, ks3, vs0,
                  ok0, ok1, ok2, ok3, ov0,
                  vt0,
                  *sems):
    del c0, c1, c2, c3, c4
    p = pos_ref[0]
    aligned = pl.multiple_of((p // 128) * 128, 128)
    col = p - aligned
    vin = pltpu.make_async_copy(ov0.at[:, :, :, pl.ds(aligned, 128)], vt0, sems[4])
    vin.start()
    k_copies = []
    for i, (ks, ok) in enumerate(((ks0, ok0), (ks1, ok1), (ks2, ok2), (ks3, ok3))):
        c = pltpu.make_async_copy(ks, ok.at[:, :, pl.ds(p, 1), :], sems[i])
        c.start()
        k_copies.append(c)
    lane = jax.lax.broadcasted_iota(jnp.int32, vt0.shape, 3)
    vin.wait()
    vt0[...] = jnp.where(lane == col, vs0[...], vt0[...])
    vout = pltpu.make_async_copy(vt0, ov0.at[:, :, :, pl.ds(aligned, 128)], sems[4])
    vout.start()
    for c in k_copies + [vout]:
        c.wait()


def kernel(input_pos, kv_cache_k_0, kv_slice_k_0, kv_cache_v_0, kv_slice_v_0, kv_cache_k_1, kv_slice_k_1, kv_cache_v_1, kv_slice_v_1, kv_cache_k_2, kv_slice_k_2, kv_cache_v_2, kv_slice_v_2, kv_cache_k_3, kv_slice_k_3, kv_cache_v_3, kv_slice_v_3):
    ov1, ov2, ov3 = _sc_update_v_caches(
        input_pos, kv_cache_v_1, kv_slice_v_1, kv_cache_v_2, kv_slice_v_2,
        kv_cache_v_3, kv_slice_v_3)

    caches = (kv_cache_k_0, kv_cache_k_1, kv_cache_k_2, kv_cache_k_3,
              kv_cache_v_0)
    slices = (kv_slice_k_0, kv_slice_k_1, kv_slice_k_2, kv_slice_k_3,
              kv_slice_v_0)

    hbm_spec = pl.BlockSpec(memory_space=pltpu.HBM)
    vmem_spec = pl.BlockSpec(memory_space=pltpu.VMEM)
    smem_spec = pl.BlockSpec(memory_space=pltpu.SMEM)
    B = 1

    out = pl.pallas_call(
        _scatter_body,
        out_shape=tuple(jax.ShapeDtypeStruct(c.shape, c.dtype) for c in caches),
        in_specs=[smem_spec] + [hbm_spec] * 5 + [vmem_spec] * 5,
        out_specs=(hbm_spec,) * 5,
        scratch_shapes=[pltpu.VMEM((B, _H, _D, 128), jnp.float32)]
                       + [pltpu.SemaphoreType.DMA] * 5,
        input_output_aliases={1 + i: i for i in range(5)},
        name="kv_cache_scatter_update",
    )(input_pos, *caches, *slices)

    ok0, ok1, ok2, ok3, ov0 = out
    return (ok0, ov0, ok1, ov1, ok2, ov2, ok3, ov3)


# speedup vs baseline: 1.0378x; 1.0378x over previous
import jax
import jax.numpy as jnp
from jax.experimental import pallas as pl
from jax.experimental.pallas import tpu as pltpu
from jax.experimental.pallas import tpu_sc as plsc

_H, _D, _KV = 4, 256, 4096
_ROWS = _H * _D
_TILES = 32
_RPT = _ROWS // _TILES
_CROWS = 16


def _sc_v3_body(pos_hbm, vs1_hbm, vc1, o1, posm, sl1, buf):
    cid = jax.lax.axis_index("c")
    sid = jax.lax.axis_index("s")
    wid = sid * 2 + cid
    pltpu.sync_copy(pos_hbm, posm)
    pltpu.sync_copy(vs1_hbm, sl1)
    p = posm[...][0]
    a16 = pl.multiple_of((p // 16) * 16, 8)
    m16 = p - a16
    ji = jax.lax.iota(jnp.int32, 16)
    for vc, sl, o in ((vc1, sl1, o1),):
        for c in range(_RPT // _CROWS):
            r = wid * _RPT + c * _CROWS
            h = r // _D
            rr = r - h * _D
            pltpu.sync_copy(vc.at[0, h, pl.ds(rr, _CROWS), :], buf)
            vals = sl[pl.ds(r, _CROWS)]
            for j in range(_CROWS):
                w = buf[j, pl.ds(a16, 16)]
                buf[j, pl.ds(a16, 16)] = jnp.where(ji == m16, vals[j], w)
            pltpu.sync_copy(buf, o.at[0, h, pl.ds(rr, _CROWS), :])


def _sc_update_v_cache(input_pos, vc1, vs1):
    run = pl.kernel(
        _sc_v3_body,
        out_type=jax.ShapeDtypeStruct(vc1.shape, vc1.dtype),
        mesh=plsc.VectorSubcoreMesh(core_axis_name="c", subcore_axis_name="s"),
        scratch_types=[
            pltpu.VMEM((16,), jnp.int32),
            pltpu.VMEM((_ROWS,), jnp.float32),
            pltpu.VMEM((_CROWS, _KV), jnp.float32),
        ],
    )
    pos16 = jnp.broadcast_to(input_pos.astype(jnp.int32), (16,))
    return run(pos16, vs1.reshape(-1), vc1)


def _scatter_body(pos_ref,
                  c0, c1, c2, c3, c4, c5, c6,
                  ks0, ks1, ks2, ks3, vs0, vs2, vs3,
                  ok0, ok1, ok2, ok3, ov0, ov2, ov3,
                  vt0, vt2, vt3,
                  *sems):
    del c0, c1, c2, c3, c4, c5, c6
    p = pos_ref[0]
    aligned = pl.multiple_of((p // 128) * 128, 128)
    col = p - aligned
    in_copies = []
    for i, (ov, vt) in enumerate(((ov0, vt0), (ov2, vt2), (ov3, vt3))):
        c = pltpu.make_async_copy(ov.at[:, :, :, pl.ds(aligned, 128)], vt, sems[4 + i])
        c.start()
        in_copies.append(c)
    k_copies = []
    for i, (ks, ok) in enumerate(((ks0, ok0), (ks1, ok1), (ks2, ok2), (ks3, ok3))):
        c = pltpu.make_async_copy(ks, ok.at[:, :, pl.ds(p, 1), :], sems[i])
        c.start()
        k_copies.append(c)
    lane = jax.lax.broadcasted_iota(jnp.int32, vt0.shape, 3)
    out_copies = []
    for i, (vs, ov, vt) in enumerate(((vs0, ov0, vt0), (vs2, ov2, vt2),
                                      (vs3, ov3, vt3))):
        in_copies[i].wait()
        vt[...] = jnp.where(lane == col, vs[...], vt[...])
        c = pltpu.make_async_copy(vt, ov.at[:, :, :, pl.ds(aligned, 128)], sems[4 + i])
        c.start()
        out_copies.append(c)
    for c in k_copies + out_copies:
        c.wait()


def kernel(input_pos, kv_cache_k_0, kv_slice_k_0, kv_cache_v_0, kv_slice_v_0, kv_cache_k_1, kv_slice_k_1, kv_cache_v_1, kv_slice_v_1, kv_cache_k_2, kv_slice_k_2, kv_cache_v_2, kv_slice_v_2, kv_cache_k_3, kv_slice_k_3, kv_cache_v_3, kv_slice_v_3):
    ov1 = _sc_update_v_cache(input_pos, kv_cache_v_1, kv_slice_v_1)

    caches = (kv_cache_k_0, kv_cache_k_1, kv_cache_k_2, kv_cache_k_3,
              kv_cache_v_0, kv_cache_v_2, kv_cache_v_3)
    slices = (kv_slice_k_0, kv_slice_k_1, kv_slice_k_2, kv_slice_k_3,
              kv_slice_v_0, kv_slice_v_2, kv_slice_v_3)

    hbm_spec = pl.BlockSpec(memory_space=pltpu.HBM)
    vmem_spec = pl.BlockSpec(memory_space=pltpu.VMEM)
    smem_spec = pl.BlockSpec(memory_space=pltpu.SMEM)
    B = 1

    out = pl.pallas_call(
        _scatter_body,
        out_shape=tuple(jax.ShapeDtypeStruct(c.shape, c.dtype) for c in caches),
        in_specs=[smem_spec] + [hbm_spec] * 7 + [vmem_spec] * 7,
        out_specs=(hbm_spec,) * 7,
        scratch_shapes=[pltpu.VMEM((B, _H, _D, 128), jnp.float32)] * 3
                       + [pltpu.SemaphoreType.DMA] * 7,
        input_output_aliases={1 + i: i for i in range(7)},
        name="kv_cache_scatter_update",
    )(input_pos, *caches, *slices)

    ok0, ok1, ok2, ok3, ov0, ov2, ov3 = out
    return (ok0, ov0, ok1, ov1, ok2, ov2, ok3, ov3)


# speedup vs baseline: 1.2087x; 1.1646x over previous
import jax
import jax.numpy as jnp
from jax.experimental import pallas as pl
from jax.experimental.pallas import tpu as pltpu


def _scatter_body(pos_ref,
                  c0, c1, c2, c3, c4, c5, c6, c7,
                  ks0, vs0, ks1, vs1, ks2, vs2, ks3, vs3,
                  ok0, ov0, ok1, ov1, ok2, ov2, ok3, ov3,
                  vt0, vt1, vt2, vt3,
                  *sems):
    del c0, c1, c2, c3, c4, c5, c6, c7
    p = pos_ref[0]
    k_copies = []
    for i, (ks, ok) in enumerate(((ks0, ok0), (ks1, ok1), (ks2, ok2), (ks3, ok3))):
        c = pltpu.make_async_copy(ks, ok.at[:, :, pl.ds(p, 1), :], sems[i])
        c.start()
        k_copies.append(c)
    aligned = pl.multiple_of((p // 128) * 128, 128)
    col = p - aligned
    in_copies = []
    for i, (ov, vt) in enumerate(((ov0, vt0), (ov1, vt1), (ov2, vt2), (ov3, vt3))):
        c = pltpu.make_async_copy(ov.at[:, :, :, pl.ds(aligned, 128)], vt, sems[4 + i])
        c.start()
        in_copies.append(c)
    lane = jax.lax.broadcasted_iota(jnp.int32, vt0.shape, 3)
    out_copies = []
    for i, (vs, ov, vt) in enumerate(((vs0, ov0, vt0), (vs1, ov1, vt1),
                                      (vs2, ov2, vt2), (vs3, ov3, vt3))):
        in_copies[i].wait()
        vt[...] = jnp.where(lane == col, vs[...], vt[...])
        c = pltpu.make_async_copy(vt, ov.at[:, :, :, pl.ds(aligned, 128)], sems[4 + i])
        c.start()
        out_copies.append(c)
    for c in k_copies + out_copies:
        c.wait()


def kernel(input_pos, kv_cache_k_0, kv_slice_k_0, kv_cache_v_0, kv_slice_v_0, kv_cache_k_1, kv_slice_k_1, kv_cache_v_1, kv_slice_v_1, kv_cache_k_2, kv_slice_k_2, kv_cache_v_2, kv_slice_v_2, kv_cache_k_3, kv_slice_k_3, kv_cache_v_3, kv_slice_v_3):
    caches = (kv_cache_k_0, kv_cache_v_0, kv_cache_k_1, kv_cache_v_1,
              kv_cache_k_2, kv_cache_v_2, kv_cache_k_3, kv_cache_v_3)
    slices = (kv_slice_k_0, kv_slice_v_0, kv_slice_k_1, kv_slice_v_1,
              kv_slice_k_2, kv_slice_v_2, kv_slice_k_3, kv_slice_v_3)

    any_spec = pl.BlockSpec(memory_space=pltpu.HBM)
    vmem_spec = pl.BlockSpec(memory_space=pltpu.VMEM)
    smem_spec = pl.BlockSpec(memory_space=pltpu.SMEM)

    out = pl.pallas_call(
        _scatter_body,
        out_shape=tuple(jax.ShapeDtypeStruct(c.shape, c.dtype) for c in caches),
        in_specs=[smem_spec] + [any_spec] * 8 + [vmem_spec] * 8,
        out_specs=(any_spec,) * 8,
        scratch_shapes=[pltpu.VMEM((1, 4, 256, 128), jnp.float32)] * 4
                       + [pltpu.SemaphoreType.DMA] * 8,
        input_output_aliases={1 + i: i for i in range(8)},
        name="kv_cache_scatter_update",
    )(input_pos, *caches, *slices)

    ok0, ov0, ok1, ov1, ok2, ov2, ok3, ov3 = out
    return (ok0, ov0, ok1, ov1, ok2, ov2, ok3, ov3)
